# Initial kernel scaffold; baseline (speedup 1.0000x reference)
import jax
import jax.numpy as jnp
from jax.experimental import pallas as pl


def kernel(x, edge_index, W_rel, b_rel, W_root, bn_weight, bn_bias):
    raise NotImplementedError("write your pallas kernel here")


# speedup vs baseline: 3.3453x; 3.3453x over previous
import functools

import jax
import jax.numpy as jnp
from jax import lax
from jax.experimental import pallas as pl
from jax.experimental.pallas import tpu as pltpu
from jax.experimental.pallas import tpu_sc as plsc

N_NODES = 10000
N_EDGES = 160000
D = 256
DH = 128
EPS = 1e-5

NC = 2
NS = 16
NPASS = 2
NODES_PER_PASS = 5120
N_PAD = NPASS * NODES_PER_PASS
GARBAGE = 64
EDGES_PER_TILE = N_EDGES // NS
CHUNK = 400
NCHUNK = EDGES_PER_TILE // CHUNK
ROWS_PER_TILE = NODES_PER_PASS // NS


_sc_mesh = plsc.VectorSubcoreMesh(
    core_axis_name="c", subcore_axis_name="s", num_cores=NC, num_subcores=NS
)


@functools.partial(
    pl.kernel,
    out_type=jax.ShapeDtypeStruct((NC * N_PAD, DH), jnp.float32),
    mesh=_sc_mesh,
    scratch_types=[
        pltpu.VMEM((EDGES_PER_TILE,), jnp.int32),
        pltpu.VMEM((EDGES_PER_TILE,), jnp.int32),
        pltpu.VMEM((EDGES_PER_TILE,), jnp.int32),
        pltpu.VMEM((CHUNK, DH), jnp.float32),
        pltpu.VMEM_SHARED((NODES_PER_PASS + GARBAGE, DH), jnp.float32),
        pltpu.SemaphoreType.DMA,
    ],
)
def _sc_agg(xcat_hbm, src_hbm, dst_hbm, zeros_hbm, out_hbm,
            src_v, dst_v, dstp_v, rows_v, agg_sh, sem):
    c = lax.axis_index("c")
    s = lax.axis_index("s")

    pltpu.sync_copy(src_hbm.at[s], src_v)
    pltpu.sync_copy(dst_hbm.at[s], dst_v)

    off = c * N_NODES

    def _src_off(k, _):
        sl = pl.ds(k * 16, 16)
        src_v[sl] = src_v[sl] + off
        return 0

    lax.fori_loop(0, EDGES_PER_TILE // 16, _src_off, 0)

    for p in range(NPASS):
        if p > 0:
            plsc.subcore_barrier()

        def _localize(k, _):
            sl = pl.ds(k * 16, 16)
            d = dst_v[sl]
            dloc = d - (p * NODES_PER_PASS)
            inr = (dloc >= 0) & (dloc < NODES_PER_PASS)
            g = NODES_PER_PASS + (d & (GARBAGE - 1))
            dstp_v[sl] = jnp.where(inr, dloc, g)
            return 0

        lax.fori_loop(0, EDGES_PER_TILE // 16, _localize, 0)

        pltpu.sync_copy(zeros_hbm, agg_sh.at[pl.ds(s * ROWS_PER_TILE, ROWS_PER_TILE)])
        plsc.subcore_barrier()

        for i in range(NCHUNK):
            idx = src_v.at[pl.ds(i * CHUNK, CHUNK)]
            pltpu.async_copy(xcat_hbm.at[idx], rows_v, sem).wait()
            didx = dstp_v.at[pl.ds(i * CHUNK, CHUNK)]
            pltpu.sync_copy(rows_v, agg_sh.at[didx], add=True)

        plsc.subcore_barrier()

        pltpu.sync_copy(
            agg_sh.at[pl.ds(s * ROWS_PER_TILE, ROWS_PER_TILE)],
            out_hbm.at[pl.ds(c * N_PAD + p * NODES_PER_PASS + s * ROWS_PER_TILE,
                             ROWS_PER_TILE)],
        )


ROWS_BLK = 1000
N_BLKS = N_NODES // ROWS_BLK


def _mm_body(aggl_ref, aggh_ref, x_ref, wr_ref, wo_ref, raw_ref, stats_ref, acc_ref):
    i = pl.program_id(0)
    r = lax.dot_general(
        x_ref[...], wo_ref[...], (((1,), (1,)), ((), ())),
        preferred_element_type=jnp.float32, precision=lax.Precision.HIGHEST,
    )
    r = r + lax.dot_general(
        aggl_ref[...], wr_ref[:, :DH], (((1,), (1,)), ((), ())),
        preferred_element_type=jnp.float32, precision=lax.Precision.HIGHEST,
    )
    r = r + lax.dot_general(
        aggh_ref[...], wr_ref[:, DH:], (((1,), (1,)), ((), ())),
        preferred_element_type=jnp.float32, precision=lax.Precision.HIGHEST,
    )
    raw_ref[...] = r
    ssum = jnp.sum(r, axis=0)
    ssq = jnp.sum(r * r, axis=0)

    @pl.when(i == 0)
    def _():
        acc_ref[0, :] = ssum
        acc_ref[1, :] = ssq

    @pl.when(i > 0)
    def _():
        acc_ref[0, :] = acc_ref[0, :] + ssum
        acc_ref[1, :] = acc_ref[1, :] + ssq

    @pl.when(i == N_BLKS - 1)
    def _():
        stats_ref[...] = acc_ref[...]


def _bn_body(raw_ref, stats_ref, bnw_ref, bnb_ref, o_ref):
    mean = stats_ref[0, :] / N_NODES
    var = stats_ref[1, :] / N_NODES - mean * mean
    scale = bnw_ref[0, :] * lax.rsqrt(var + EPS)
    shift = bnb_ref[0, :] - mean * scale
    y = raw_ref[...] * scale[None, :] + shift[None, :]
    o_ref[...] = jnp.where(y >= 0, y, 0.1 * y)


def kernel(x, edge_index, W_rel, b_rel, W_root, bn_weight, bn_bias):
    del b_rel
    src = edge_index[0].astype(jnp.int32).reshape(NS, EDGES_PER_TILE)
    dst = edge_index[1].astype(jnp.int32).reshape(NS, EDGES_PER_TILE)
    xcat = jnp.concatenate([x[:, :DH], x[:, DH:]], axis=0)
    zeros = jnp.zeros((ROWS_PER_TILE, DH), jnp.float32)

    agg_cat = _sc_agg(xcat, src, dst, zeros)
    agg_lo = agg_cat[:N_NODES]
    agg_hi = agg_cat[N_PAD:N_PAD + N_NODES]

    raw, stats = pl.pallas_call(
        _mm_body,
        grid=(N_BLKS,),
        in_specs=[
            pl.BlockSpec((ROWS_BLK, DH), lambda i: (i, 0)),
            pl.BlockSpec((ROWS_BLK, DH), lambda i: (i, 0)),
            pl.BlockSpec((ROWS_BLK, D), lambda i: (i, 0)),
            pl.BlockSpec((D, D), lambda i: (0, 0)),
            pl.BlockSpec((D, D), lambda i: (0, 0)),
        ],
        out_specs=[
            pl.BlockSpec((ROWS_BLK, D), lambda i: (i, 0)),
            pl.BlockSpec((2, D), lambda i: (0, 0)),
        ],
        out_shape=[
            jax.ShapeDtypeStruct((N_NODES, D), jnp.float32),
            jax.ShapeDtypeStruct((2, D), jnp.float32),
        ],
        scratch_shapes=[pltpu.VMEM((2, D), jnp.float32)],
    )(agg_lo, agg_hi, x, W_rel, W_root)

    out = pl.pallas_call(
        _bn_body,
        grid=(N_BLKS,),
        in_specs=[
            pl.BlockSpec((ROWS_BLK, D), lambda i: (i, 0)),
            pl.BlockSpec((2, D), lambda i: (0, 0)),
            pl.BlockSpec((1, D), lambda i: (0, 0)),
            pl.BlockSpec((1, D), lambda i: (0, 0)),
        ],
        out_specs=pl.BlockSpec((ROWS_BLK, D), lambda i: (i, 0)),
        out_shape=jax.ShapeDtypeStruct((N_NODES, D), jnp.float32),
    )(raw, stats, bn_weight.reshape(1, D), bn_bias.reshape(1, D))

    return out


# speedup vs baseline: 3.8417x; 1.1484x over previous
import functools

import jax
import jax.numpy as jnp
from jax import lax
from jax.experimental import pallas as pl
from jax.experimental.pallas import tpu as pltpu
from jax.experimental.pallas import tpu_sc as plsc

N_NODES = 10000
N_EDGES = 160000
D = 256
DH = 128
EPS = 1e-5

NC = 2
NS = 16
NHALF = 5120
N_PAD = 2 * NHALF
GARBAGE = 64
EDGES_PER_TILE = N_EDGES // NS
CH_E = 2000
N_STAGE = EDGES_PER_TILE // CH_E
CHUNK = 80
NPAIR = CH_E // (2 * CHUNK)
ROWS_PER_TILE = NHALF // NS


_sc_mesh = plsc.VectorSubcoreMesh(
    core_axis_name="c", subcore_axis_name="s", num_cores=NC, num_subcores=NS
)


@functools.partial(
    pl.kernel,
    out_type=jax.ShapeDtypeStruct((NC * N_PAD, DH), jnp.float32),
    mesh=_sc_mesh,
    scratch_types=[
        pltpu.VMEM((CH_E,), jnp.int32),
        pltpu.VMEM((CH_E,), jnp.int32),
        pltpu.VMEM((CH_E,), jnp.int32),
        pltpu.VMEM((CHUNK, DH), jnp.float32),
        pltpu.VMEM((CHUNK, DH), jnp.float32),
        pltpu.VMEM_SHARED((NHALF + GARBAGE, DH), jnp.float32),
        pltpu.VMEM_SHARED((NHALF + GARBAGE, DH), jnp.float32),
        pltpu.SemaphoreType.DMA,
        pltpu.SemaphoreType.DMA,
    ],
)
def _sc_agg(xcat_hbm, src_hbm, dst_hbm, zeros_hbm, out_hbm,
            src_c, dsta_c, dstb_c, rows_a, rows_b, agg_a, agg_b,
            sem_a, sem_b):
    c = lax.axis_index("c")
    s = lax.axis_index("s")

    goff = c * N_NODES

    own = pl.ds(s * ROWS_PER_TILE, ROWS_PER_TILE)
    pltpu.sync_copy(zeros_hbm, agg_a.at[own])
    pltpu.sync_copy(zeros_hbm, agg_b.at[own])
    plsc.subcore_barrier()

    for st in range(N_STAGE):
        eb = s * EDGES_PER_TILE + st * CH_E
        pltpu.sync_copy(src_hbm.at[pl.ds(eb, CH_E)], src_c)
        pltpu.sync_copy(dst_hbm.at[pl.ds(eb, CH_E)], dsta_c)

        def _localize(k, _):
            sl = pl.ds(k * 16, 16)
            d = dsta_c[sl]
            g = NHALF + (d & (GARBAGE - 1))
            db = d - NHALF
            dstb_c[sl] = jnp.where(db >= 0, db, g)
            dsta_c[sl] = jnp.where(d < NHALF, d, g)
            src_c[sl] = src_c[sl] + goff
            return 0

        lax.fori_loop(0, CH_E // 16, _localize, 0)

        def _chunk_scatter(off, buf):
            pltpu.sync_copy(buf, agg_a.at[dsta_c.at[pl.ds(off, CHUNK)]], add=True)
            pltpu.sync_copy(buf, agg_b.at[dstb_c.at[pl.ds(off, CHUNK)]], add=True)

        def _pair(j, _):
            b0 = j * (2 * CHUNK)
            b1 = b0 + CHUNK
            cp_a = pltpu.async_copy(
                xcat_hbm.at[src_c.at[pl.ds(b0, CHUNK)]], rows_a, sem_a)
            cp_b = pltpu.async_copy(
                xcat_hbm.at[src_c.at[pl.ds(b1, CHUNK)]], rows_b, sem_b)
            cp_a.wait()
            _chunk_scatter(b0, rows_a)
            cp_b.wait()
            _chunk_scatter(b1, rows_b)
            return 0

        lax.fori_loop(0, NPAIR, _pair, 0)

        tb = NPAIR * 2 * CHUNK
        pltpu.async_copy(
            xcat_hbm.at[src_c.at[pl.ds(tb, CHUNK)]], rows_a, sem_a).wait()
        _chunk_scatter(tb, rows_a)

    plsc.subcore_barrier()

    pltpu.sync_copy(agg_a.at[own],
                    out_hbm.at[pl.ds(c * N_PAD + s * ROWS_PER_TILE, ROWS_PER_TILE)])
    pltpu.sync_copy(agg_b.at[own],
                    out_hbm.at[pl.ds(c * N_PAD + NHALF + s * ROWS_PER_TILE,
                                     ROWS_PER_TILE)])


ROWS_BLK = 1000
N_BLKS = N_NODES // ROWS_BLK


def _mm_body(aggl_ref, aggh_ref, x_ref, wr_ref, wo_ref, raw_ref, stats_ref, acc_ref):
    i = pl.program_id(0)
    r = lax.dot_general(
        x_ref[...], wo_ref[...], (((1,), (1,)), ((), ())),
        preferred_element_type=jnp.float32, precision=lax.Precision.HIGHEST,
    )
    r = r + lax.dot_general(
        aggl_ref[...], wr_ref[:, :DH], (((1,), (1,)), ((), ())),
        preferred_element_type=jnp.float32, precision=lax.Precision.HIGHEST,
    )
    r = r + lax.dot_general(
        aggh_ref[...], wr_ref[:, DH:], (((1,), (1,)), ((), ())),
        preferred_element_type=jnp.float32, precision=lax.Precision.HIGHEST,
    )
    raw_ref[...] = r
    ssum = jnp.sum(r, axis=0)
    ssq = jnp.sum(r * r, axis=0)

    @pl.when(i == 0)
    def _():
        acc_ref[0, :] = ssum
        acc_ref[1, :] = ssq

    @pl.when(i > 0)
    def _():
        acc_ref[0, :] = acc_ref[0, :] + ssum
        acc_ref[1, :] = acc_ref[1, :] + ssq

    @pl.when(i == N_BLKS - 1)
    def _():
        stats_ref[...] = acc_ref[...]


def _bn_body(raw_ref, stats_ref, bnw_ref, bnb_ref, o_ref):
    mean = stats_ref[0, :] / N_NODES
    var = stats_ref[1, :] / N_NODES - mean * mean
    scale = bnw_ref[0, :] * lax.rsqrt(var + EPS)
    shift = bnb_ref[0, :] - mean * scale
    y = raw_ref[...] * scale[None, :] + shift[None, :]
    o_ref[...] = jnp.where(y >= 0, y, 0.1 * y)


def kernel(x, edge_index, W_rel, b_rel, W_root, bn_weight, bn_bias):
    del b_rel
    src = edge_index[0].astype(jnp.int32)
    dst = edge_index[1].astype(jnp.int32)
    xcat = jnp.concatenate([x[:, :DH], x[:, DH:]], axis=0)
    zeros = jnp.zeros((ROWS_PER_TILE, DH), jnp.float32)

    agg_cat = _sc_agg(xcat, src, dst, zeros)
    agg_lo = agg_cat[:N_NODES]
    agg_hi = agg_cat[N_PAD:N_PAD + N_NODES]

    raw, stats = pl.pallas_call(
        _mm_body,
        grid=(N_BLKS,),
        in_specs=[
            pl.BlockSpec((ROWS_BLK, DH), lambda i: (i, 0)),
            pl.BlockSpec((ROWS_BLK, DH), lambda i: (i, 0)),
            pl.BlockSpec((ROWS_BLK, D), lambda i: (i, 0)),
            pl.BlockSpec((D, D), lambda i: (0, 0)),
            pl.BlockSpec((D, D), lambda i: (0, 0)),
        ],
        out_specs=[
            pl.BlockSpec((ROWS_BLK, D), lambda i: (i, 0)),
            pl.BlockSpec((2, D), lambda i: (0, 0)),
        ],
        out_shape=[
            jax.ShapeDtypeStruct((N_NODES, D), jnp.float32),
            jax.ShapeDtypeStruct((2, D), jnp.float32),
        ],
        scratch_shapes=[pltpu.VMEM((2, D), jnp.float32)],
    )(agg_lo, agg_hi, x, W_rel, W_root)

    out = pl.pallas_call(
        _bn_body,
        grid=(N_BLKS,),
        in_specs=[
            pl.BlockSpec((ROWS_BLK, D), lambda i: (i, 0)),
            pl.BlockSpec((2, D), lambda i: (0, 0)),
            pl.BlockSpec((1, D), lambda i: (0, 0)),
            pl.BlockSpec((1, D), lambda i: (0, 0)),
        ],
        out_specs=pl.BlockSpec((ROWS_BLK, D), lambda i: (i, 0)),
        out_shape=jax.ShapeDtypeStruct((N_NODES, D), jnp.float32),
    )(raw, stats, bn_weight.reshape(1, D), bn_bias.reshape(1, D))

    return out


# speedup vs baseline: 4.7603x; 1.2391x over previous
import functools

import jax
import jax.numpy as jnp
from jax import lax
from jax.experimental import pallas as pl
from jax.experimental.pallas import tpu as pltpu
from jax.experimental.pallas import tpu_sc as plsc

N_NODES = 10000
N_EDGES = 160000
D = 256
DH = 128
EPS = 1e-5

NC = 2
NS = 16
N_PAD = 10240
EDGES_PER_TILE = N_EDGES // NS
CH_E = 2000
N_STAGE = EDGES_PER_TILE // CH_E
CHUNK = 80
NPAIR = CH_E // (2 * CHUNK)
ROWS_PER_TILE = N_PAD // NS


_sc_mesh = plsc.VectorSubcoreMesh(
    core_axis_name="c", subcore_axis_name="s", num_cores=NC, num_subcores=NS
)


@functools.partial(
    pl.kernel,
    out_type=jax.ShapeDtypeStruct((NC * N_PAD, DH), jnp.float32),
    mesh=_sc_mesh,
    scratch_types=[
        pltpu.VMEM((CH_E,), jnp.int32),
        pltpu.VMEM((CH_E,), jnp.int32),
        pltpu.VMEM((CHUNK, DH), jnp.float32),
        pltpu.VMEM((CHUNK, DH), jnp.float32),
        pltpu.VMEM_SHARED((N_PAD, DH), jnp.float32),
        pltpu.SemaphoreType.DMA,
        pltpu.SemaphoreType.DMA,
    ],
)
def _sc_agg(xcat_hbm, src_hbm, dst_hbm, zeros_hbm, out_hbm,
            src_c, dst_c, rows_a, rows_b, agg_sh, sem_a, sem_b):
    c = lax.axis_index("c")
    s = lax.axis_index("s")

    goff = c * N_NODES

    own = pl.ds(s * ROWS_PER_TILE, ROWS_PER_TILE)
    pltpu.sync_copy(zeros_hbm, agg_sh.at[own])
    plsc.subcore_barrier()

    for st in range(N_STAGE):
        eb = s * EDGES_PER_TILE + st * CH_E
        pltpu.sync_copy(src_hbm.at[pl.ds(eb, CH_E)], src_c)
        pltpu.sync_copy(dst_hbm.at[pl.ds(eb, CH_E)], dst_c)

        def _src_off(k, _):
            sl = pl.ds(k * 16, 16)
            src_c[sl] = src_c[sl] + goff
            return 0

        lax.fori_loop(0, CH_E // 16, _src_off, 0)

        def _pair(j, _):
            b0 = j * (2 * CHUNK)
            b1 = b0 + CHUNK
            cp_a = pltpu.async_copy(
                xcat_hbm.at[src_c.at[pl.ds(b0, CHUNK)]], rows_a, sem_a)
            cp_b = pltpu.async_copy(
                xcat_hbm.at[src_c.at[pl.ds(b1, CHUNK)]], rows_b, sem_b)
            cp_a.wait()
            pltpu.sync_copy(rows_a, agg_sh.at[dst_c.at[pl.ds(b0, CHUNK)]], add=True)
            cp_b.wait()
            pltpu.sync_copy(rows_b, agg_sh.at[dst_c.at[pl.ds(b1, CHUNK)]], add=True)
            return 0

        lax.fori_loop(0, NPAIR, _pair, 0)

        tb = NPAIR * 2 * CHUNK
        pltpu.async_copy(
            xcat_hbm.at[src_c.at[pl.ds(tb, CHUNK)]], rows_a, sem_a).wait()
        pltpu.sync_copy(rows_a, agg_sh.at[dst_c.at[pl.ds(tb, CHUNK)]], add=True)

    plsc.subcore_barrier()

    pltpu.sync_copy(agg_sh.at[own],
                    out_hbm.at[pl.ds(c * N_PAD + s * ROWS_PER_TILE, ROWS_PER_TILE)])


ROWS_BLK = 1000
N_BLKS = N_NODES // ROWS_BLK


def _mm_body(aggl_ref, aggh_ref, x_ref, wr_ref, wo_ref, raw_ref, stats_ref, acc_ref):
    i = pl.program_id(0)
    r = lax.dot_general(
        x_ref[...], wo_ref[...], (((1,), (1,)), ((), ())),
        preferred_element_type=jnp.float32, precision=lax.Precision.HIGHEST,
    )
    r = r + lax.dot_general(
        aggl_ref[...], wr_ref[:, :DH], (((1,), (1,)), ((), ())),
        preferred_element_type=jnp.float32, precision=lax.Precision.HIGHEST,
    )
    r = r + lax.dot_general(
        aggh_ref[...], wr_ref[:, DH:], (((1,), (1,)), ((), ())),
        preferred_element_type=jnp.float32, precision=lax.Precision.HIGHEST,
    )
    raw_ref[...] = r
    ssum = jnp.sum(r, axis=0)
    ssq = jnp.sum(r * r, axis=0)

    @pl.when(i == 0)
    def _():
        acc_ref[0, :] = ssum
        acc_ref[1, :] = ssq

    @pl.when(i > 0)
    def _():
        acc_ref[0, :] = acc_ref[0, :] + ssum
        acc_ref[1, :] = acc_ref[1, :] + ssq

    @pl.when(i == N_BLKS - 1)
    def _():
        stats_ref[...] = acc_ref[...]


def _bn_body(raw_ref, stats_ref, bnw_ref, bnb_ref, o_ref):
    mean = stats_ref[0, :] / N_NODES
    var = stats_ref[1, :] / N_NODES - mean * mean
    scale = bnw_ref[0, :] * lax.rsqrt(var + EPS)
    shift = bnb_ref[0, :] - mean * scale
    y = raw_ref[...] * scale[None, :] + shift[None, :]
    o_ref[...] = jnp.where(y >= 0, y, 0.1 * y)


def kernel(x, edge_index, W_rel, b_rel, W_root, bn_weight, bn_bias):
    del b_rel
    src = edge_index[0].astype(jnp.int32)
    dst = edge_index[1].astype(jnp.int32)
    xcat = jnp.concatenate([x[:, :DH], x[:, DH:]], axis=0)
    zeros = jnp.zeros((ROWS_PER_TILE, DH), jnp.float32)

    agg_cat = _sc_agg(xcat, src, dst, zeros)
    agg_lo = agg_cat[:N_NODES]
    agg_hi = agg_cat[N_PAD:N_PAD + N_NODES]

    raw, stats = pl.pallas_call(
        _mm_body,
        grid=(N_BLKS,),
        in_specs=[
            pl.BlockSpec((ROWS_BLK, DH), lambda i: (i, 0)),
            pl.BlockSpec((ROWS_BLK, DH), lambda i: (i, 0)),
            pl.BlockSpec((ROWS_BLK, D), lambda i: (i, 0)),
            pl.BlockSpec((D, D), lambda i: (0, 0)),
            pl.BlockSpec((D, D), lambda i: (0, 0)),
        ],
        out_specs=[
            pl.BlockSpec((ROWS_BLK, D), lambda i: (i, 0)),
            pl.BlockSpec((2, D), lambda i: (0, 0)),
        ],
        out_shape=[
            jax.ShapeDtypeStruct((N_NODES, D), jnp.float32),
            jax.ShapeDtypeStruct((2, D), jnp.float32),
        ],
        scratch_shapes=[pltpu.VMEM((2, D), jnp.float32)],
    )(agg_lo, agg_hi, x, W_rel, W_root)

    out = pl.pallas_call(
        _bn_body,
        grid=(N_BLKS,),
        in_specs=[
            pl.BlockSpec((ROWS_BLK, D), lambda i: (i, 0)),
            pl.BlockSpec((2, D), lambda i: (0, 0)),
            pl.BlockSpec((1, D), lambda i: (0, 0)),
            pl.BlockSpec((1, D), lambda i: (0, 0)),
        ],
        out_specs=pl.BlockSpec((ROWS_BLK, D), lambda i: (i, 0)),
        out_shape=jax.ShapeDtypeStruct((N_NODES, D), jnp.float32),
    )(raw, stats, bn_weight.reshape(1, D), bn_bias.reshape(1, D))

    return out


# speedup vs baseline: 5.3496x; 1.1238x over previous
import functools

import jax
import jax.numpy as jnp
from jax import lax
from jax.experimental import pallas as pl
from jax.experimental.pallas import tpu as pltpu
from jax.experimental.pallas import tpu_sc as plsc

N_NODES = 10000
N_EDGES = 160000
D = 256
DH = 128
EPS = 1e-5

NC = 2
NS = 16
N_PAD = 10240
EDGES_PER_TILE = N_EDGES // NS
CH_E = 2000
N_STAGE = EDGES_PER_TILE // CH_E
CHUNK = 80
NPAIR = CH_E // (2 * CHUNK)
ROWS_PER_TILE = N_PAD // NS


_sc_mesh = plsc.VectorSubcoreMesh(
    core_axis_name="c", subcore_axis_name="s", num_cores=NC, num_subcores=NS
)


@functools.partial(
    pl.kernel,
    out_type=[jax.ShapeDtypeStruct((N_PAD, DH), jnp.float32),
              jax.ShapeDtypeStruct((N_PAD, DH), jnp.float32)],
    mesh=_sc_mesh,
    scratch_types=[
        pltpu.VMEM((CH_E,), jnp.int32),
        pltpu.VMEM((CH_E,), jnp.int32),
        pltpu.VMEM((CHUNK, DH), jnp.float32),
        pltpu.VMEM((CHUNK, DH), jnp.float32),
        pltpu.VMEM_SHARED((N_PAD, DH), jnp.float32),
        pltpu.SemaphoreType.DMA,
        pltpu.SemaphoreType.DMA,
    ],
)
def _sc_agg(x_hbm, src_hbm, dst_hbm, zeros_hbm, out_lo, out_hi,
            src_c, dst_c, rows_a, rows_b, agg_sh, sem_a, sem_b):
    c = lax.axis_index("c")
    s = lax.axis_index("s")

    col = pl.ds(c * DH, DH)

    own = pl.ds(s * ROWS_PER_TILE, ROWS_PER_TILE)
    pltpu.sync_copy(zeros_hbm, agg_sh.at[own])
    plsc.subcore_barrier()

    for st in range(N_STAGE):
        eb = s * EDGES_PER_TILE + st * CH_E
        pltpu.sync_copy(src_hbm.at[pl.ds(eb, CH_E)], src_c)
        pltpu.sync_copy(dst_hbm.at[pl.ds(eb, CH_E)], dst_c)

        def _pair(j, _):
            b0 = j * (2 * CHUNK)
            b1 = b0 + CHUNK
            cp_a = pltpu.async_copy(
                x_hbm.at[src_c.at[pl.ds(b0, CHUNK)], col], rows_a, sem_a)
            cp_b = pltpu.async_copy(
                x_hbm.at[src_c.at[pl.ds(b1, CHUNK)], col], rows_b, sem_b)
            cp_a.wait()
            pltpu.sync_copy(rows_a, agg_sh.at[dst_c.at[pl.ds(b0, CHUNK)]], add=True)
            cp_b.wait()
            pltpu.sync_copy(rows_b, agg_sh.at[dst_c.at[pl.ds(b1, CHUNK)]], add=True)
            return 0

        lax.fori_loop(0, NPAIR, _pair, 0)

        tb = NPAIR * 2 * CHUNK
        pltpu.async_copy(
            x_hbm.at[src_c.at[pl.ds(tb, CHUNK)], col], rows_a, sem_a).wait()
        pltpu.sync_copy(rows_a, agg_sh.at[dst_c.at[pl.ds(tb, CHUNK)]], add=True)

    plsc.subcore_barrier()

    @pl.when(c == 0)
    def _():
        pltpu.sync_copy(agg_sh.at[own], out_lo.at[own])

    @pl.when(c == 1)
    def _():
        pltpu.sync_copy(agg_sh.at[own], out_hi.at[own])


ROWS_BLK = 1000
N_BLKS = N_NODES // ROWS_BLK


def _mm_body(aggl_ref, aggh_ref, x_ref, wr_ref, wo_ref, raw_ref, stats_ref, acc_ref):
    i = pl.program_id(0)
    r = lax.dot_general(
        x_ref[...], wo_ref[...], (((1,), (1,)), ((), ())),
        preferred_element_type=jnp.float32, precision=lax.Precision.HIGHEST,
    )
    r = r + lax.dot_general(
        aggl_ref[...], wr_ref[:, :DH], (((1,), (1,)), ((), ())),
        preferred_element_type=jnp.float32, precision=lax.Precision.HIGHEST,
    )
    r = r + lax.dot_general(
        aggh_ref[...], wr_ref[:, DH:], (((1,), (1,)), ((), ())),
        preferred_element_type=jnp.float32, precision=lax.Precision.HIGHEST,
    )
    raw_ref[...] = r
    ssum = jnp.sum(r, axis=0)
    ssq = jnp.sum(r * r, axis=0)

    @pl.when(i == 0)
    def _():
        acc_ref[0, :] = ssum
        acc_ref[1, :] = ssq

    @pl.when(i > 0)
    def _():
        acc_ref[0, :] = acc_ref[0, :] + ssum
        acc_ref[1, :] = acc_ref[1, :] + ssq

    @pl.when(i == N_BLKS - 1)
    def _():
        stats_ref[...] = acc_ref[...]


def _bn_body(raw_ref, stats_ref, bnw_ref, bnb_ref, o_ref):
    mean = stats_ref[0, :] / N_NODES
    var = stats_ref[1, :] / N_NODES - mean * mean
    scale = bnw_ref[0, :] * lax.rsqrt(var + EPS)
    shift = bnb_ref[0, :] - mean * scale
    y = raw_ref[...] * scale[None, :] + shift[None, :]
    o_ref[...] = jnp.where(y >= 0, y, 0.1 * y)


def kernel(x, edge_index, W_rel, b_rel, W_root, bn_weight, bn_bias):
    del b_rel
    src = edge_index[0].astype(jnp.int32)
    dst = edge_index[1].astype(jnp.int32)
    zeros = jnp.zeros((ROWS_PER_TILE, DH), jnp.float32)

    agg_lo, agg_hi = _sc_agg(x, src, dst, zeros)

    raw, stats = pl.pallas_call(
        _mm_body,
        grid=(N_BLKS,),
        in_specs=[
            pl.BlockSpec((ROWS_BLK, DH), lambda i: (i, 0)),
            pl.BlockSpec((ROWS_BLK, DH), lambda i: (i, 0)),
            pl.BlockSpec((ROWS_BLK, D), lambda i: (i, 0)),
            pl.BlockSpec((D, D), lambda i: (0, 0)),
            pl.BlockSpec((D, D), lambda i: (0, 0)),
        ],
        out_specs=[
            pl.BlockSpec((ROWS_BLK, D), lambda i: (i, 0)),
            pl.BlockSpec((2, D), lambda i: (0, 0)),
        ],
        out_shape=[
            jax.ShapeDtypeStruct((N_NODES, D), jnp.float32),
            jax.ShapeDtypeStruct((2, D), jnp.float32),
        ],
        scratch_shapes=[pltpu.VMEM((2, D), jnp.float32)],
    )(agg_lo, agg_hi, x, W_rel, W_root)

    out = pl.pallas_call(
        _bn_body,
        grid=(N_BLKS,),
        in_specs=[
            pl.BlockSpec((ROWS_BLK, D), lambda i: (i, 0)),
            pl.BlockSpec((2, D), lambda i: (0, 0)),
            pl.BlockSpec((1, D), lambda i: (0, 0)),
            pl.BlockSpec((1, D), lambda i: (0, 0)),
        ],
        out_specs=pl.BlockSpec((ROWS_BLK, D), lambda i: (i, 0)),
        out_shape=jax.ShapeDtypeStruct((N_NODES, D), jnp.float32),
    )(raw, stats, bn_weight.reshape(1, D), bn_bias.reshape(1, D))

    return out


# speedup vs baseline: 5.6623x; 1.0584x over previous
import functools

import jax
import jax.numpy as jnp
from jax import lax
from jax.experimental import pallas as pl
from jax.experimental.pallas import tpu as pltpu
from jax.experimental.pallas import tpu_sc as plsc

N_NODES = 10000
N_EDGES = 160000
D = 256
DH = 128
EPS = 1e-5

NC = 2
NS = 16
N_PAD = 10240
EDGES_PER_TILE = N_EDGES // NS
CH_E = 2000
N_STAGE = EDGES_PER_TILE // CH_E
CHUNK = 80
NPAIR = CH_E // (2 * CHUNK)
ROWS_PER_TILE = N_PAD // NS


_sc_mesh = plsc.VectorSubcoreMesh(
    core_axis_name="c", subcore_axis_name="s", num_cores=NC, num_subcores=NS
)


@functools.partial(
    pl.kernel,
    out_type=[jax.ShapeDtypeStruct((N_PAD, DH), jnp.float32),
              jax.ShapeDtypeStruct((N_PAD, DH), jnp.float32)],
    mesh=_sc_mesh,
    scratch_types=[
        pltpu.VMEM((CH_E,), jnp.int32),
        pltpu.VMEM((CH_E,), jnp.int32),
        pltpu.VMEM((CHUNK, DH), jnp.float32),
        pltpu.VMEM((CHUNK, DH), jnp.float32),
        pltpu.VMEM_SHARED((N_PAD, DH), jnp.float32),
        pltpu.SemaphoreType.DMA,
        pltpu.SemaphoreType.DMA,
    ],
)
def _sc_agg(x_hbm, src_hbm, dst_hbm, zeros_hbm, out_lo, out_hi,
            src_c, dst_c, rows_a, rows_b, agg_sh, sem_a, sem_b):
    c = lax.axis_index("c")
    s = lax.axis_index("s")

    col = pl.ds(c * DH, DH)

    own = pl.ds(s * ROWS_PER_TILE, ROWS_PER_TILE)
    pltpu.sync_copy(zeros_hbm, agg_sh.at[own])
    plsc.subcore_barrier()

    for st in range(N_STAGE):
        eb = s * EDGES_PER_TILE + st * CH_E
        pltpu.sync_copy(src_hbm.at[pl.ds(eb, CH_E)], src_c)
        pltpu.sync_copy(dst_hbm.at[pl.ds(eb, CH_E)], dst_c)

        def _pair(j, _):
            b0 = j * (2 * CHUNK)
            b1 = b0 + CHUNK
            cp_a = pltpu.async_copy(
                x_hbm.at[src_c.at[pl.ds(b0, CHUNK)], col], rows_a, sem_a)
            cp_b = pltpu.async_copy(
                x_hbm.at[src_c.at[pl.ds(b1, CHUNK)], col], rows_b, sem_b)
            cp_a.wait()
            pltpu.sync_copy(rows_a, agg_sh.at[dst_c.at[pl.ds(b0, CHUNK)]], add=True)
            cp_b.wait()
            pltpu.sync_copy(rows_b, agg_sh.at[dst_c.at[pl.ds(b1, CHUNK)]], add=True)
            return 0

        lax.fori_loop(0, NPAIR, _pair, 0)

        tb = NPAIR * 2 * CHUNK
        pltpu.async_copy(
            x_hbm.at[src_c.at[pl.ds(tb, CHUNK)], col], rows_a, sem_a).wait()
        pltpu.sync_copy(rows_a, agg_sh.at[dst_c.at[pl.ds(tb, CHUNK)]], add=True)

    plsc.subcore_barrier()

    @pl.when(c == 0)
    def _():
        pltpu.sync_copy(agg_sh.at[own], out_lo.at[own])

    @pl.when(c == 1)
    def _():
        pltpu.sync_copy(agg_sh.at[own], out_hi.at[own])


ROWS_BLK = 1000
N_BLKS = N_NODES // ROWS_BLK


def _mm1_body(x_ref, wo_ref, raw0_ref):
    raw0_ref[...] = lax.dot_general(
        x_ref[...], wo_ref[...], (((1,), (1,)), ((), ())),
        preferred_element_type=jnp.float32, precision=lax.Precision.DEFAULT,
    )


def _mm2_body(raw0_ref, aggl_ref, aggh_ref, wr_ref, raw_ref, stats_ref, acc_ref):
    i = pl.program_id(0)
    r = raw0_ref[...] + lax.dot_general(
        aggl_ref[...], wr_ref[:, :DH], (((1,), (1,)), ((), ())),
        preferred_element_type=jnp.float32, precision=lax.Precision.DEFAULT,
    )
    r = r + lax.dot_general(
        aggh_ref[...], wr_ref[:, DH:], (((1,), (1,)), ((), ())),
        preferred_element_type=jnp.float32, precision=lax.Precision.DEFAULT,
    )
    raw_ref[...] = r
    ssum = jnp.sum(r, axis=0)
    ssq = jnp.sum(r * r, axis=0)

    @pl.when(i == 0)
    def _():
        acc_ref[0, :] = ssum
        acc_ref[1, :] = ssq

    @pl.when(i > 0)
    def _():
        acc_ref[0, :] = acc_ref[0, :] + ssum
        acc_ref[1, :] = acc_ref[1, :] + ssq

    @pl.when(i == N_BLKS - 1)
    def _():
        stats_ref[...] = acc_ref[...]


def _bn_body(raw_ref, stats_ref, bnw_ref, bnb_ref, o_ref):
    mean = stats_ref[0, :] / N_NODES
    var = stats_ref[1, :] / N_NODES - mean * mean
    scale = bnw_ref[0, :] * lax.rsqrt(var + EPS)
    shift = bnb_ref[0, :] - mean * scale
    y = raw_ref[...] * scale[None, :] + shift[None, :]
    o_ref[...] = jnp.where(y >= 0, y, 0.1 * y)


def kernel(x, edge_index, W_rel, b_rel, W_root, bn_weight, bn_bias):
    del b_rel
    src = edge_index[0].astype(jnp.int32)
    dst = edge_index[1].astype(jnp.int32)
    zeros = jnp.zeros((ROWS_PER_TILE, DH), jnp.float32)

    raw0 = pl.pallas_call(
        _mm1_body,
        grid=(N_BLKS,),
        in_specs=[
            pl.BlockSpec((ROWS_BLK, D), lambda i: (i, 0)),
            pl.BlockSpec((D, D), lambda i: (0, 0)),
        ],
        out_specs=pl.BlockSpec((ROWS_BLK, D), lambda i: (i, 0)),
        out_shape=jax.ShapeDtypeStruct((N_NODES, D), jnp.float32),
    )(x, W_root)

    agg_lo, agg_hi = _sc_agg(x, src, dst, zeros)

    raw, stats = pl.pallas_call(
        _mm2_body,
        grid=(N_BLKS,),
        in_specs=[
            pl.BlockSpec((ROWS_BLK, D), lambda i: (i, 0)),
            pl.BlockSpec((ROWS_BLK, DH), lambda i: (i, 0)),
            pl.BlockSpec((ROWS_BLK, DH), lambda i: (i, 0)),
            pl.BlockSpec((D, D), lambda i: (0, 0)),
        ],
        out_specs=[
            pl.BlockSpec((ROWS_BLK, D), lambda i: (i, 0)),
            pl.BlockSpec((2, D), lambda i: (0, 0)),
        ],
        out_shape=[
            jax.ShapeDtypeStruct((N_NODES, D), jnp.float32),
            jax.ShapeDtypeStruct((2, D), jnp.float32),
        ],
        scratch_shapes=[pltpu.VMEM((2, D), jnp.float32)],
    )(raw0, agg_lo, agg_hi, W_rel)

    out = pl.pallas_call(
        _bn_body,
        grid=(N_BLKS,),
        in_specs=[
            pl.BlockSpec((ROWS_BLK, D), lambda i: (i, 0)),
            pl.BlockSpec((2, D), lambda i: (0, 0)),
            pl.BlockSpec((1, D), lambda i: (0, 0)),
            pl.BlockSpec((1, D), lambda i: (0, 0)),
        ],
        out_specs=pl.BlockSpec((ROWS_BLK, D), lambda i: (i, 0)),
        out_shape=jax.ShapeDtypeStruct((N_NODES, D), jnp.float32),
    )(raw, stats, bn_weight.reshape(1, D), bn_bias.reshape(1, D))

    return out


# speedup vs baseline: 6.2901x; 1.1109x over previous
import functools

import jax
import jax.numpy as jnp
from jax import lax
from jax.experimental import pallas as pl
from jax.experimental.pallas import tpu as pltpu
from jax.experimental.pallas import tpu_sc as plsc

N_NODES = 10000
N_EDGES = 160000
D = 256
DH = 128
EPS = 1e-5

NC = 2
NS = 16
EDGES_PER_TILE = N_EDGES // NS
CH_E = 2000
N_STAGE = EDGES_PER_TILE // CH_E
CHUNK = 80
NCHUNK = CH_E // CHUNK
NRING = 4
ROWS_PER_TILE = 640
TAIL_ROWS = N_NODES - 15 * ROWS_PER_TILE


_sc_mesh = plsc.VectorSubcoreMesh(
    core_axis_name="c", subcore_axis_name="s", num_cores=NC, num_subcores=NS
)


@functools.partial(
    pl.kernel,
    out_type=[jax.ShapeDtypeStruct((N_NODES, DH), jnp.float32),
              jax.ShapeDtypeStruct((N_NODES, DH), jnp.float32)],
    mesh=_sc_mesh,
    scratch_types=[
        [pltpu.VMEM((CH_E,), jnp.int32) for _ in range(2)],
        [pltpu.VMEM((CH_E,), jnp.int32) for _ in range(2)],
        [pltpu.VMEM((CHUNK, DH), jnp.float32) for _ in range(NRING)],
        pltpu.VMEM_SHARED((N_NODES, DH), jnp.float32),
        [pltpu.SemaphoreType.DMA for _ in range(NRING)],
        [pltpu.SemaphoreType.DMA for _ in range(2)],
    ],
)
def _sc_agg(x_hbm, src_hbm, dst_hbm, zeros_hbm, out_lo, out_hi,
            src_b, dst_b, rows, agg_sh, gsem, ssem):
    c = lax.axis_index("c")
    s = lax.axis_index("s")

    col = pl.ds(c * DH, DH)

    def _stage_copies(st, k):
        eb = s * EDGES_PER_TILE + st * CH_E
        return (pltpu.async_copy(src_hbm.at[pl.ds(eb, CH_E)], src_b[k], ssem[0]),
                pltpu.async_copy(dst_hbm.at[pl.ds(eb, CH_E)], dst_b[k], ssem[1]))

    st_cp = _stage_copies(0, 0)

    @pl.when(s < 15)
    def _():
        pltpu.sync_copy(zeros_hbm,
                        agg_sh.at[pl.ds(s * ROWS_PER_TILE, ROWS_PER_TILE)])

    @pl.when(s == 15)
    def _():
        pltpu.sync_copy(zeros_hbm.at[pl.ds(0, TAIL_ROWS)],
                        agg_sh.at[pl.ds(15 * ROWS_PER_TILE, TAIL_ROWS)])

    st_cp[0].wait()
    st_cp[1].wait()
    plsc.subcore_barrier()

    for st in range(N_STAGE):
        cur = st % 2
        if st > 0:
            cp0, cp1 = _stage_copies(st, cur)
            cp0.wait()
            cp1.wait()
        src_c = src_b[cur]
        dst_c = dst_b[cur]

        def _gather(off, i):
            return pltpu.async_copy(
                x_hbm.at[src_c.at[pl.ds(off, CHUNK)], col],
                rows[i], gsem[i])

        def _quad(j, _):
            b = j * (NRING * CHUNK)
            cps = [_gather(b + i * CHUNK, i) for i in range(NRING)]
            for i in range(NRING):
                cps[i].wait()
                pltpu.sync_copy(
                    rows[i],
                    agg_sh.at[dst_c.at[pl.ds(b + i * CHUNK, CHUNK)]], add=True)
            return 0

        lax.fori_loop(0, NCHUNK // NRING, _quad, 0)

        tb = (NCHUNK // NRING) * NRING * CHUNK
        _gather(tb, 0).wait()
        pltpu.sync_copy(rows[0],
                        agg_sh.at[dst_c.at[pl.ds(tb, CHUNK)]], add=True)


    plsc.subcore_barrier()

    def _writeout(dst_hbm_ref):
        @pl.when(s < 15)
        def _():
            own = pl.ds(s * ROWS_PER_TILE, ROWS_PER_TILE)
            pltpu.sync_copy(agg_sh.at[own], dst_hbm_ref.at[own])

        @pl.when(s == 15)
        def _():
            own = pl.ds(15 * ROWS_PER_TILE, TAIL_ROWS)
            pltpu.sync_copy(agg_sh.at[own], dst_hbm_ref.at[own])

    @pl.when(c == 0)
    def _():
        _writeout(out_lo)

    @pl.when(c == 1)
    def _():
        _writeout(out_hi)


ROWS_BLK = 1000
N_BLKS = N_NODES // ROWS_BLK


def _mm1_body(x_ref, wo_ref, raw0_ref):
    raw0_ref[...] = lax.dot_general(
        x_ref[...], wo_ref[...], (((1,), (1,)), ((), ())),
        preferred_element_type=jnp.float32, precision=lax.Precision.DEFAULT,
    )


def _mm2_body(raw0_ref, aggl_ref, aggh_ref, wr_ref, raw_ref, stats_ref, acc_ref):
    i = pl.program_id(0)
    r = raw0_ref[...] + lax.dot_general(
        aggl_ref[...], wr_ref[:, :DH], (((1,), (1,)), ((), ())),
        preferred_element_type=jnp.float32, precision=lax.Precision.DEFAULT,
    )
    r = r + lax.dot_general(
        aggh_ref[...], wr_ref[:, DH:], (((1,), (1,)), ((), ())),
        preferred_element_type=jnp.float32, precision=lax.Precision.DEFAULT,
    )
    raw_ref[...] = r
    ssum = jnp.sum(r, axis=0)
    ssq = jnp.sum(r * r, axis=0)

    @pl.when(i == 0)
    def _():
        acc_ref[0, :] = ssum
        acc_ref[1, :] = ssq

    @pl.when(i > 0)
    def _():
        acc_ref[0, :] = acc_ref[0, :] + ssum
        acc_ref[1, :] = acc_ref[1, :] + ssq

    @pl.when(i == N_BLKS - 1)
    def _():
        stats_ref[...] = acc_ref[...]


def _bn_body(raw_ref, stats_ref, bnw_ref, bnb_ref, o_ref):
    mean = stats_ref[0, :] / N_NODES
    var = stats_ref[1, :] / N_NODES - mean * mean
    scale = bnw_ref[0, :] * lax.rsqrt(var + EPS)
    shift = bnb_ref[0, :] - mean * scale
    y = raw_ref[...] * scale[None, :] + shift[None, :]
    o_ref[...] = jnp.where(y >= 0, y, 0.1 * y)


def kernel(x, edge_index, W_rel, b_rel, W_root, bn_weight, bn_bias):
    del b_rel
    src = edge_index[0].astype(jnp.int32)
    dst = edge_index[1].astype(jnp.int32)
    zeros = jnp.zeros((ROWS_PER_TILE, DH), jnp.float32)

    raw0 = pl.pallas_call(
        _mm1_body,
        grid=(N_BLKS,),
        in_specs=[
            pl.BlockSpec((ROWS_BLK, D), lambda i: (i, 0)),
            pl.BlockSpec((D, D), lambda i: (0, 0)),
        ],
        out_specs=pl.BlockSpec((ROWS_BLK, D), lambda i: (i, 0)),
        out_shape=jax.ShapeDtypeStruct((N_NODES, D), jnp.float32),
    )(x, W_root)

    agg_lo, agg_hi = _sc_agg(x, src, dst, zeros)

    raw, stats = pl.pallas_call(
        _mm2_body,
        grid=(N_BLKS,),
        in_specs=[
            pl.BlockSpec((ROWS_BLK, D), lambda i: (i, 0)),
            pl.BlockSpec((ROWS_BLK, DH), lambda i: (i, 0)),
            pl.BlockSpec((ROWS_BLK, DH), lambda i: (i, 0)),
            pl.BlockSpec((D, D), lambda i: (0, 0)),
        ],
        out_specs=[
            pl.BlockSpec((ROWS_BLK, D), lambda i: (i, 0)),
            pl.BlockSpec((2, D), lambda i: (0, 0)),
        ],
        out_shape=[
            jax.ShapeDtypeStruct((N_NODES, D), jnp.float32),
            jax.ShapeDtypeStruct((2, D), jnp.float32),
        ],
        scratch_shapes=[pltpu.VMEM((2, D), jnp.float32)],
    )(raw0, agg_lo, agg_hi, W_rel)

    out = pl.pallas_call(
        _bn_body,
        grid=(N_BLKS,),
        in_specs=[
            pl.BlockSpec((ROWS_BLK, D), lambda i: (i, 0)),
            pl.BlockSpec((2, D), lambda i: (0, 0)),
            pl.BlockSpec((1, D), lambda i: (0, 0)),
            pl.BlockSpec((1, D), lambda i: (0, 0)),
        ],
        out_specs=pl.BlockSpec((ROWS_BLK, D), lambda i: (i, 0)),
        out_shape=jax.ShapeDtypeStruct((N_NODES, D), jnp.float32),
    )(raw, stats, bn_weight.reshape(1, D), bn_bias.reshape(1, D))

    return out


# speedup vs baseline: 7.1107x; 1.1305x over previous
import functools

import jax
import jax.numpy as jnp
from jax import lax
from jax.experimental import pallas as pl
from jax.experimental.pallas import tpu as pltpu
from jax.experimental.pallas import tpu_sc as plsc

N_NODES = 10000
N_EDGES = 160000
D = 256
DH = 128
EPS = 1e-5

NC = 2
NS = 16
EDGES_PER_TILE = N_EDGES // NS
CH_E = 2000
N_STAGE = EDGES_PER_TILE // CH_E
CHUNK = 80
NCHUNK = CH_E // CHUNK
NRING = 4
ROWS_PER_TILE = 640
TAIL_ROWS = N_NODES - 15 * ROWS_PER_TILE


_sc_mesh = plsc.VectorSubcoreMesh(
    core_axis_name="c", subcore_axis_name="s", num_cores=NC, num_subcores=NS
)


@functools.partial(
    pl.kernel,
    out_type=[jax.ShapeDtypeStruct((N_NODES, DH), jnp.float32),
              jax.ShapeDtypeStruct((N_NODES, DH), jnp.float32)],
    mesh=_sc_mesh,
    scratch_types=[
        pltpu.VMEM((CH_E,), jnp.int32),
        pltpu.VMEM((CH_E,), jnp.int32),
        [pltpu.VMEM((CHUNK, DH), jnp.float32) for _ in range(NRING)],
        pltpu.VMEM_SHARED((N_NODES, DH), jnp.float32),
        [pltpu.SemaphoreType.DMA for _ in range(NRING)],
        [pltpu.SemaphoreType.DMA for _ in range(2)],
    ],
)
def _sc_agg(x_hbm, src_hbm, dst_hbm, zeros_hbm, out_lo, out_hi,
            src_c, dst_c, rows, agg_sh, gsem, ssem):
    c = lax.axis_index("c")
    s = lax.axis_index("s")

    col = pl.ds(c * DH, DH)

    @pl.when(s < 15)
    def _():
        pltpu.sync_copy(zeros_hbm,
                        agg_sh.at[pl.ds(s * ROWS_PER_TILE, ROWS_PER_TILE)])

    @pl.when(s == 15)
    def _():
        pltpu.sync_copy(zeros_hbm.at[pl.ds(0, TAIL_ROWS)],
                        agg_sh.at[pl.ds(15 * ROWS_PER_TILE, TAIL_ROWS)])

    plsc.subcore_barrier()

    def _issue(off, i):
        return pltpu.async_copy(
            x_hbm.at[src_c.at[pl.ds(off, CHUNK)], col], rows[i], gsem[i])

    def _wait(i):
        pltpu.make_async_copy(
            x_hbm.at[src_c.at[pl.ds(0, CHUNK)], col], rows[i], gsem[i]).wait()

    def _scatter(off, i):
        pltpu.sync_copy(rows[i],
                        agg_sh.at[dst_c.at[pl.ds(off, CHUNK)]], add=True)

    def _stage(st, _):
        eb = s * EDGES_PER_TILE + st * CH_E
        pltpu.sync_copy(src_hbm.at[pl.ds(eb, CH_E)], src_c)
        pltpu.sync_copy(dst_hbm.at[pl.ds(eb, CH_E)], dst_c)

        for i in range(NRING):
            _issue(i * CHUNK, i)

        def _quad(j, _):
            for i in range(NRING):
                ch = j * NRING + i
                _wait(i)
                _scatter(ch * CHUNK, i)

                @pl.when(ch + NRING < NCHUNK)
                def _():
                    _issue((ch + NRING) * CHUNK, i)
            return 0

        lax.fori_loop(0, NCHUNK // NRING, _quad, 0)

        _wait(0)
        _scatter((NCHUNK - 1) * CHUNK, 0)
        return 0

    lax.fori_loop(0, N_STAGE, _stage, 0)

    plsc.subcore_barrier()

    def _writeout(dst_hbm_ref):
        @pl.when(s < 15)
        def _():
            own = pl.ds(s * ROWS_PER_TILE, ROWS_PER_TILE)
            pltpu.sync_copy(agg_sh.at[own], dst_hbm_ref.at[own])

        @pl.when(s == 15)
        def _():
            own = pl.ds(15 * ROWS_PER_TILE, TAIL_ROWS)
            pltpu.sync_copy(agg_sh.at[own], dst_hbm_ref.at[own])

    @pl.when(c == 0)
    def _():
        _writeout(out_lo)

    @pl.when(c == 1)
    def _():
        _writeout(out_hi)


ROWS_BLK = 1000
N_BLKS = N_NODES // ROWS_BLK


def _mm1_body(x_ref, wo_ref, raw0_ref):
    raw0_ref[...] = lax.dot_general(
        x_ref[...], wo_ref[...], (((1,), (1,)), ((), ())),
        preferred_element_type=jnp.float32, precision=lax.Precision.DEFAULT,
    )


def _mm2_body(raw0_ref, aggl_ref, aggh_ref, wr_ref, raw_ref, stats_ref, acc_ref):
    i = pl.program_id(0)
    r = raw0_ref[...] + lax.dot_general(
        aggl_ref[...], wr_ref[:, :DH], (((1,), (1,)), ((), ())),
        preferred_element_type=jnp.float32, precision=lax.Precision.DEFAULT,
    )
    r = r + lax.dot_general(
        aggh_ref[...], wr_ref[:, DH:], (((1,), (1,)), ((), ())),
        preferred_element_type=jnp.float32, precision=lax.Precision.DEFAULT,
    )
    raw_ref[...] = r
    ssum = jnp.sum(r, axis=0)
    ssq = jnp.sum(r * r, axis=0)

    @pl.when(i == 0)
    def _():
        acc_ref[0, :] = ssum
        acc_ref[1, :] = ssq

    @pl.when(i > 0)
    def _():
        acc_ref[0, :] = acc_ref[0, :] + ssum
        acc_ref[1, :] = acc_ref[1, :] + ssq

    @pl.when(i == N_BLKS - 1)
    def _():
        stats_ref[...] = acc_ref[...]


def _bn_body(raw_ref, stats_ref, bnw_ref, bnb_ref, o_ref):
    mean = stats_ref[0, :] / N_NODES
    var = stats_ref[1, :] / N_NODES - mean * mean
    scale = bnw_ref[0, :] * lax.rsqrt(var + EPS)
    shift = bnb_ref[0, :] - mean * scale
    y = raw_ref[...] * scale[None, :] + shift[None, :]
    o_ref[...] = jnp.where(y >= 0, y, 0.1 * y)


def kernel(x, edge_index, W_rel, b_rel, W_root, bn_weight, bn_bias):
    del b_rel
    src = edge_index[0].astype(jnp.int32)
    dst = edge_index[1].astype(jnp.int32)
    zeros = jnp.zeros((ROWS_PER_TILE, DH), jnp.float32)

    raw0 = pl.pallas_call(
        _mm1_body,
        grid=(N_BLKS,),
        in_specs=[
            pl.BlockSpec((ROWS_BLK, D), lambda i: (i, 0)),
            pl.BlockSpec((D, D), lambda i: (0, 0)),
        ],
        out_specs=pl.BlockSpec((ROWS_BLK, D), lambda i: (i, 0)),
        out_shape=jax.ShapeDtypeStruct((N_NODES, D), jnp.float32),
    )(x, W_root)

    agg_lo, agg_hi = _sc_agg(x, src, dst, zeros)

    raw, stats = pl.pallas_call(
        _mm2_body,
        grid=(N_BLKS,),
        in_specs=[
            pl.BlockSpec((ROWS_BLK, D), lambda i: (i, 0)),
            pl.BlockSpec((ROWS_BLK, DH), lambda i: (i, 0)),
            pl.BlockSpec((ROWS_BLK, DH), lambda i: (i, 0)),
            pl.BlockSpec((D, D), lambda i: (0, 0)),
        ],
        out_specs=[
            pl.BlockSpec((ROWS_BLK, D), lambda i: (i, 0)),
            pl.BlockSpec((2, D), lambda i: (0, 0)),
        ],
        out_shape=[
            jax.ShapeDtypeStruct((N_NODES, D), jnp.float32),
            jax.ShapeDtypeStruct((2, D), jnp.float32),
        ],
        scratch_shapes=[pltpu.VMEM((2, D), jnp.float32)],
    )(raw0, agg_lo, agg_hi, W_rel)

    out = pl.pallas_call(
        _bn_body,
        grid=(N_BLKS,),
        in_specs=[
            pl.BlockSpec((ROWS_BLK, D), lambda i: (i, 0)),
            pl.BlockSpec((2, D), lambda i: (0, 0)),
            pl.BlockSpec((1, D), lambda i: (0, 0)),
            pl.BlockSpec((1, D), lambda i: (0, 0)),
        ],
        out_specs=pl.BlockSpec((ROWS_BLK, D), lambda i: (i, 0)),
        out_shape=jax.ShapeDtypeStruct((N_NODES, D), jnp.float32),
    )(raw, stats, bn_weight.reshape(1, D), bn_bias.reshape(1, D))

    return out


# speedup vs baseline: 7.2138x; 1.0145x over previous
import functools

import jax
import jax.numpy as jnp
from jax import lax
from jax.experimental import pallas as pl
from jax.experimental.pallas import tpu as pltpu
from jax.experimental.pallas import tpu_sc as plsc

N_NODES = 10000
N_EDGES = 160000
D = 256
DH = 128
EPS = 1e-5

NC = 2
NS = 16
EDGES_PER_TILE = N_EDGES // NS
CH_E = 2000
N_STAGE = EDGES_PER_TILE // CH_E
CHUNK = 80
NCHUNK = CH_E // CHUNK
NRING = 4
ROWS_PER_TILE = 640
TAIL_ROWS = N_NODES - 15 * ROWS_PER_TILE


_sc_mesh = plsc.VectorSubcoreMesh(
    core_axis_name="c", subcore_axis_name="s", num_cores=NC, num_subcores=NS
)


@functools.partial(
    pl.kernel,
    out_type=[jax.ShapeDtypeStruct((N_NODES, DH), jnp.float32),
              jax.ShapeDtypeStruct((N_NODES, DH), jnp.float32)],
    mesh=_sc_mesh,
    scratch_types=[
        pltpu.VMEM((CH_E,), jnp.int32),
        pltpu.VMEM((CH_E,), jnp.int32),
        [pltpu.VMEM((CHUNK, DH), jnp.float32) for _ in range(NRING)],
        pltpu.VMEM_SHARED((N_NODES, DH), jnp.float32),
        [pltpu.SemaphoreType.DMA for _ in range(NRING)],
        [pltpu.SemaphoreType.DMA for _ in range(2)],
    ],
)
def _sc_agg(x_hbm, src_hbm, dst_hbm, zeros_hbm, out_lo, out_hi,
            src_c, dst_c, rows, agg_sh, gsem, ssem):
    c = lax.axis_index("c")
    s = lax.axis_index("s")

    col = pl.ds(c * DH, DH)

    @pl.when(s < 15)
    def _():
        pltpu.sync_copy(zeros_hbm,
                        agg_sh.at[pl.ds(s * ROWS_PER_TILE, ROWS_PER_TILE)])

    @pl.when(s == 15)
    def _():
        pltpu.sync_copy(zeros_hbm.at[pl.ds(0, TAIL_ROWS)],
                        agg_sh.at[pl.ds(15 * ROWS_PER_TILE, TAIL_ROWS)])

    plsc.subcore_barrier()

    def _issue(off, i):
        return pltpu.async_copy(
            x_hbm.at[src_c.at[pl.ds(off, CHUNK)], col], rows[i], gsem[i])

    def _wait(i):
        pltpu.make_async_copy(
            x_hbm.at[src_c.at[pl.ds(0, CHUNK)], col], rows[i], gsem[i]).wait()

    def _scatter(off, i):
        pltpu.sync_copy(rows[i],
                        agg_sh.at[dst_c.at[pl.ds(off, CHUNK)]], add=True)

    def _stage(st, _):
        eb = s * EDGES_PER_TILE + st * CH_E
        pltpu.sync_copy(src_hbm.at[pl.ds(eb, CH_E)], src_c)
        pltpu.sync_copy(dst_hbm.at[pl.ds(eb, CH_E)], dst_c)

        for i in range(NRING):
            _issue(i * CHUNK, i)

        def _quad(j, _):
            for i in range(NRING):
                ch = j * NRING + i
                _wait(i)
                _scatter(ch * CHUNK, i)

                @pl.when(ch + NRING < NCHUNK)
                def _():
                    _issue((ch + NRING) * CHUNK, i)
            return 0

        lax.fori_loop(0, NCHUNK // NRING, _quad, 0)

        _wait(0)
        _scatter((NCHUNK - 1) * CHUNK, 0)
        return 0

    lax.fori_loop(0, N_STAGE, _stage, 0)

    plsc.subcore_barrier()

    def _writeout(dst_hbm_ref):
        @pl.when(s < 15)
        def _():
            own = pl.ds(s * ROWS_PER_TILE, ROWS_PER_TILE)
            pltpu.sync_copy(agg_sh.at[own], dst_hbm_ref.at[own])

        @pl.when(s == 15)
        def _():
            own = pl.ds(15 * ROWS_PER_TILE, TAIL_ROWS)
            pltpu.sync_copy(agg_sh.at[own], dst_hbm_ref.at[own])

    @pl.when(c == 0)
    def _():
        _writeout(out_lo)

    @pl.when(c == 1)
    def _():
        _writeout(out_hi)


ROWS_BLK = 1000
N_BLKS = N_NODES // ROWS_BLK


def _mm_body(x_ref, aggl_ref, aggh_ref, wr_ref, wo_ref, raw_ref, stats_ref, acc_ref):
    i = pl.program_id(0)
    r = lax.dot_general(
        x_ref[...], wo_ref[...], (((1,), (1,)), ((), ())),
        preferred_element_type=jnp.float32, precision=lax.Precision.DEFAULT,
    )
    r = r + lax.dot_general(
        aggl_ref[...], wr_ref[:, :DH], (((1,), (1,)), ((), ())),
        preferred_element_type=jnp.float32, precision=lax.Precision.DEFAULT,
    )
    r = r + lax.dot_general(
        aggh_ref[...], wr_ref[:, DH:], (((1,), (1,)), ((), ())),
        preferred_element_type=jnp.float32, precision=lax.Precision.DEFAULT,
    )
    raw_ref[...] = r
    ssum = jnp.sum(r, axis=0)
    ssq = jnp.sum(r * r, axis=0)

    @pl.when(i == 0)
    def _():
        acc_ref[0, :] = ssum
        acc_ref[1, :] = ssq

    @pl.when(i > 0)
    def _():
        acc_ref[0, :] = acc_ref[0, :] + ssum
        acc_ref[1, :] = acc_ref[1, :] + ssq

    @pl.when(i == N_BLKS - 1)
    def _():
        stats_ref[...] = acc_ref[...]


def _bn_body(raw_ref, stats_ref, bnw_ref, bnb_ref, o_ref):
    mean = stats_ref[0, :] / N_NODES
    var = stats_ref[1, :] / N_NODES - mean * mean
    scale = bnw_ref[0, :] * lax.rsqrt(var + EPS)
    shift = bnb_ref[0, :] - mean * scale
    y = raw_ref[...] * scale[None, :] + shift[None, :]
    o_ref[...] = jnp.where(y >= 0, y, 0.1 * y)


def kernel(x, edge_index, W_rel, b_rel, W_root, bn_weight, bn_bias):
    del b_rel
    src = edge_index[0].astype(jnp.int32)
    dst = edge_index[1].astype(jnp.int32)
    zeros = jnp.zeros((ROWS_PER_TILE, DH), jnp.float32)

    agg_lo, agg_hi = _sc_agg(x, src, dst, zeros)

    raw, stats = pl.pallas_call(
        _mm_body,
        grid=(N_BLKS,),
        in_specs=[
            pl.BlockSpec((ROWS_BLK, D), lambda i: (i, 0)),
            pl.BlockSpec((ROWS_BLK, DH), lambda i: (i, 0)),
            pl.BlockSpec((ROWS_BLK, DH), lambda i: (i, 0)),
            pl.BlockSpec((D, D), lambda i: (0, 0)),
            pl.BlockSpec((D, D), lambda i: (0, 0)),
        ],
        out_specs=[
            pl.BlockSpec((ROWS_BLK, D), lambda i: (i, 0)),
            pl.BlockSpec((2, D), lambda i: (0, 0)),
        ],
        out_shape=[
            jax.ShapeDtypeStruct((N_NODES, D), jnp.float32),
            jax.ShapeDtypeStruct((2, D), jnp.float32),
        ],
        scratch_shapes=[pltpu.VMEM((2, D), jnp.float32)],
    )(x, agg_lo, agg_hi, W_rel, W_root)

    out = pl.pallas_call(
        _bn_body,
        grid=(N_BLKS,),
        in_specs=[
            pl.BlockSpec((ROWS_BLK, D), lambda i: (i, 0)),
            pl.BlockSpec((2, D), lambda i: (0, 0)),
            pl.BlockSpec((1, D), lambda i: (0, 0)),
            pl.BlockSpec((1, D), lambda i: (0, 0)),
        ],
        out_specs=pl.BlockSpec((ROWS_BLK, D), lambda i: (i, 0)),
        out_shape=jax.ShapeDtypeStruct((N_NODES, D), jnp.float32),
    )(raw, stats, bn_weight.reshape(1, D), bn_bias.reshape(1, D))

    return out


# speedup vs baseline: 7.5559x; 1.0474x over previous
import functools

import jax
import jax.numpy as jnp
from jax import lax
from jax.experimental import pallas as pl
from jax.experimental.pallas import tpu as pltpu
from jax.experimental.pallas import tpu_sc as plsc

N_NODES = 10000
N_EDGES = 160000
D = 256
DH = 128
EPS = 1e-5

NC = 2
NS = 16
EDGES_PER_TILE = N_EDGES // NS
CH_E = 2000
N_STAGE = EDGES_PER_TILE // CH_E
CHUNK = 80
NCHUNK = CH_E // CHUNK
NRING = 4
ROWS_PER_TILE = 640
TAIL_ROWS = N_NODES - 15 * ROWS_PER_TILE


_sc_mesh = plsc.VectorSubcoreMesh(
    core_axis_name="c", subcore_axis_name="s", num_cores=NC, num_subcores=NS
)


@functools.partial(
    pl.kernel,
    out_type=[jax.ShapeDtypeStruct((N_NODES, DH), jnp.float32),
              jax.ShapeDtypeStruct((N_NODES, DH), jnp.float32)],
    mesh=_sc_mesh,
    scratch_types=[
        pltpu.VMEM((CH_E,), jnp.int32),
        pltpu.VMEM((CH_E,), jnp.int32),
        [pltpu.VMEM((CHUNK, DH), jnp.float32) for _ in range(NRING)],
        pltpu.VMEM_SHARED((N_NODES, DH), jnp.float32),
        [pltpu.SemaphoreType.DMA for _ in range(NRING)],
        [pltpu.SemaphoreType.DMA for _ in range(2)],
    ],
)
def _sc_agg(x_hbm, src_hbm, dst_hbm, zeros_hbm, out_lo, out_hi,
            src_c, dst_c, rows, agg_sh, gsem, ssem):
    c = lax.axis_index("c")
    s = lax.axis_index("s")

    col = pl.ds(c * DH, DH)

    @pl.when(s < 15)
    def _():
        pltpu.sync_copy(zeros_hbm,
                        agg_sh.at[pl.ds(s * ROWS_PER_TILE, ROWS_PER_TILE)])

    @pl.when(s == 15)
    def _():
        pltpu.sync_copy(zeros_hbm.at[pl.ds(0, TAIL_ROWS)],
                        agg_sh.at[pl.ds(15 * ROWS_PER_TILE, TAIL_ROWS)])

    plsc.subcore_barrier()

    def _issue(off, i):
        return pltpu.async_copy(
            x_hbm.at[src_c.at[pl.ds(off, CHUNK)], col], rows[i], gsem[i])

    def _wait(i):
        pltpu.make_async_copy(
            x_hbm.at[src_c.at[pl.ds(0, CHUNK)], col], rows[i], gsem[i]).wait()

    def _scatter(off, i):
        pltpu.sync_copy(rows[i],
                        agg_sh.at[dst_c.at[pl.ds(off, CHUNK)]], add=True)

    def _stage(st, _):
        eb = s * EDGES_PER_TILE + st * CH_E
        pltpu.sync_copy(src_hbm.at[pl.ds(eb, CH_E)], src_c)
        pltpu.sync_copy(dst_hbm.at[pl.ds(eb, CH_E)], dst_c)

        for i in range(NRING):
            _issue(i * CHUNK, i)

        def _quad(j, _):
            for i in range(NRING):
                ch = j * NRING + i
                _wait(i)
                _scatter(ch * CHUNK, i)

                @pl.when(ch + NRING < NCHUNK)
                def _():
                    _issue((ch + NRING) * CHUNK, i)
            return 0

        lax.fori_loop(0, NCHUNK // NRING, _quad, 0)

        _wait(0)
        _scatter((NCHUNK - 1) * CHUNK, 0)
        return 0

    lax.fori_loop(0, N_STAGE, _stage, 0)

    plsc.subcore_barrier()

    def _writeout(dst_hbm_ref):
        @pl.when(s < 15)
        def _():
            own = pl.ds(s * ROWS_PER_TILE, ROWS_PER_TILE)
            pltpu.sync_copy(agg_sh.at[own], dst_hbm_ref.at[own])

        @pl.when(s == 15)
        def _():
            own = pl.ds(15 * ROWS_PER_TILE, TAIL_ROWS)
            pltpu.sync_copy(agg_sh.at[own], dst_hbm_ref.at[own])

    @pl.when(c == 0)
    def _():
        _writeout(out_lo)

    @pl.when(c == 1)
    def _():
        _writeout(out_hi)


def _tc_body(x_ref, aggl_ref, aggh_ref, wr_ref, wo_ref, bnw_ref, bnb_ref, o_ref):
    r = lax.dot_general(
        x_ref[...], wo_ref[...], (((1,), (1,)), ((), ())),
        preferred_element_type=jnp.float32, precision=lax.Precision.DEFAULT,
    )
    r = r + lax.dot_general(
        aggl_ref[...], wr_ref[:, :DH], (((1,), (1,)), ((), ())),
        preferred_element_type=jnp.float32, precision=lax.Precision.DEFAULT,
    )
    r = r + lax.dot_general(
        aggh_ref[...], wr_ref[:, DH:], (((1,), (1,)), ((), ())),
        preferred_element_type=jnp.float32, precision=lax.Precision.DEFAULT,
    )
    mean = jnp.sum(r, axis=0) / N_NODES
    var = jnp.sum(r * r, axis=0) / N_NODES - mean * mean
    scale = bnw_ref[0, :] * lax.rsqrt(var + EPS)
    shift = bnb_ref[0, :] - mean * scale
    y = r * scale[None, :] + shift[None, :]
    o_ref[...] = jnp.where(y >= 0, y, 0.1 * y)


def kernel(x, edge_index, W_rel, b_rel, W_root, bn_weight, bn_bias):
    del b_rel
    src = edge_index[0].astype(jnp.int32)
    dst = edge_index[1].astype(jnp.int32)
    zeros = jnp.zeros((ROWS_PER_TILE, DH), jnp.float32)

    agg_lo, agg_hi = _sc_agg(x, src, dst, zeros)

    out = pl.pallas_call(
        _tc_body,
        in_specs=[
            pl.BlockSpec((N_NODES, D), lambda: (0, 0)),
            pl.BlockSpec((N_NODES, DH), lambda: (0, 0)),
            pl.BlockSpec((N_NODES, DH), lambda: (0, 0)),
            pl.BlockSpec((D, D), lambda: (0, 0)),
            pl.BlockSpec((D, D), lambda: (0, 0)),
            pl.BlockSpec((1, D), lambda: (0, 0)),
            pl.BlockSpec((1, D), lambda: (0, 0)),
        ],
        out_specs=pl.BlockSpec((N_NODES, D), lambda: (0, 0)),
        out_shape=jax.ShapeDtypeStruct((N_NODES, D), jnp.float32),
    )(x, agg_lo, agg_hi, W_rel, W_root,
      bn_weight.reshape(1, D), bn_bias.reshape(1, D))

    return out
